# Initial kernel scaffold; baseline (speedup 1.0000x reference)
import jax
import jax.numpy as jnp
from jax.experimental import pallas as pl


def kernel(cls_logits, bbox_pred, anchors, image_h, image_w):
    raise NotImplementedError("write your pallas kernel here")


# speedup vs baseline: 10.1346x; 10.1346x over previous
import functools

import jax
import jax.numpy as jnp
from jax.experimental import pallas as pl
from jax.experimental.pallas import tpu as pltpu

_CV = 0.1
_SV = 0.2
_CONF = 0.01
_NMS_T = 0.5
_MAXOUT = 100
_N = 20000
_ROWS = 160
_LANES = 128
_TILES = _ROWS // 8
_NEG = -1e9


def _nms_body(l0, l1, tcx, tcy, tw, th, acx, acy, aw, ah, dims, out_ref,
              sc_ref, x0_ref, y0_ref, x1_ref, y1_ref):
    sw = dims[0, 0]
    sh = dims[0, 1]

    a = l0[...]
    b = l1[...]
    mx = jnp.maximum(a, b)
    e0 = jnp.exp(a - mx)
    e1 = jnp.exp(b - mx)
    fg = e1 / (e0 + e1)

    cx = tcx[...] * _CV * aw[...] + acx[...]
    cy = tcy[...] * _CV * ah[...] + acy[...]
    w = jnp.exp(tw[...] * _SV) * aw[...]
    h = jnp.exp(th[...] * _SV) * ah[...]
    x0 = (cx - w / 2.0) * sw
    y0 = (cy - h / 2.0) * sh
    x1 = (cx + w / 2.0) * sw
    y1 = (cy + h / 2.0) * sh

    row_i = jax.lax.broadcasted_iota(jnp.int32, (_ROWS, _LANES), 0)
    lane_i = jax.lax.broadcasted_iota(jnp.int32, (_ROWS, _LANES), 1)
    gidx = row_i * _LANES + lane_i
    masked = jnp.where((fg > _CONF) & (gidx < _N), fg, _NEG)

    sc_ref[...] = masked
    x0_ref[...] = x0
    y0_ref[...] = y0
    x1_ref[...] = x1
    y1_ref[...] = y1

    lane1 = jax.lax.broadcasted_iota(jnp.int32, (1, _LANES), 1)
    tmax = jnp.full((1, _LANES), _NEG, jnp.float32)
    for t in range(_TILES):
        mt = jnp.max(masked[t * 8:(t + 1) * 8, :])
        tmax = jnp.where(lane1 == t, mt, tmax)

    fill_x0 = jnp.full((1, _LANES), x0[0, 0], jnp.float32)
    fill_y0 = jnp.full((1, _LANES), y0[0, 0], jnp.float32)
    fill_x1 = jnp.full((1, _LANES), x1[0, 0], jnp.float32)
    fill_y1 = jnp.full((1, _LANES), y1[0, 0], jnp.float32)
    fill_sc = jnp.full((1, _LANES), fg[0, 0], jnp.float32)

    flat_i = jax.lax.broadcasted_iota(jnp.int32, (8, _LANES), 0) * _LANES + \
        jax.lax.broadcasted_iota(jnp.int32, (8, _LANES), 1)

    def cond(state):
        k, m, _, _, _, _, _, _ = state
        return jnp.logical_and(k < _MAXOUT, m > _NEG)

    def body(state):
        k, m, tmax, sx0, sy0, sx1, sy1, ssc = state
        t = jnp.min(jnp.where(tmax == m, lane1, jnp.int32(10 ** 9)))
        start = pl.multiple_of(t * 8, 8)
        tile = sc_ref[pl.ds(start, 8), :]
        cflat = jnp.min(jnp.where(tile == m, flat_i, jnp.int32(10 ** 9)))
        hit = flat_i == cflat
        cx0 = jnp.max(jnp.where(hit, x0_ref[pl.ds(start, 8), :], _NEG))
        cy0 = jnp.max(jnp.where(hit, y0_ref[pl.ds(start, 8), :], _NEG))
        cx1 = jnp.max(jnp.where(hit, x1_ref[pl.ds(start, 8), :], _NEG))
        cy1 = jnp.max(jnp.where(hit, y1_ref[pl.ds(start, 8), :], _NEG))

        iw = jnp.maximum(jnp.minimum(cx1, sx1) - jnp.maximum(cx0, sx0), 0.0)
        ih = jnp.maximum(jnp.minimum(cy1, sy1) - jnp.maximum(cy0, sy0), 0.0)
        inter = iw * ih
        a1 = jnp.maximum(cx1 - cx0, 0.0) * jnp.maximum(cy1 - cy0, 0.0)
        a2 = jnp.maximum(sx1 - sx0, 0.0) * jnp.maximum(sy1 - sy0, 0.0)
        iou = inter / (a1 + a2 - inter + 1e-9)
        supp = jnp.max(jnp.where(lane1 < k, iou, 0.0)) > _NMS_T

        put = jnp.logical_and(jnp.logical_not(supp), lane1 == k)
        sx0 = jnp.where(put, cx0, sx0)
        sy0 = jnp.where(put, cy0, sy0)
        sx1 = jnp.where(put, cx1, sx1)
        sy1 = jnp.where(put, cy1, sy1)
        ssc = jnp.where(put, m, ssc)
        k = k + jnp.where(supp, 0, 1).astype(jnp.int32)

        ntile = jnp.where(hit, _NEG, tile)
        sc_ref[pl.ds(start, 8), :] = ntile
        tmax = jnp.where(lane1 == t, jnp.max(ntile), tmax)
        m = jnp.max(tmax)
        return k, m, tmax, sx0, sy0, sx1, sy1, ssc

    m0 = jnp.max(tmax)
    init = (jnp.int32(0), m0, tmax, fill_x0, fill_y0, fill_x1, fill_y1,
            fill_sc)
    _, _, _, sx0, sy0, sx1, sy1, ssc = jax.lax.while_loop(cond, body, init)

    out_ref[...] = jnp.concatenate(
        [sx0, sy0, sx1, sy1, ssc, jnp.zeros((3, _LANES), jnp.float32)], axis=0)


def _plane(x):
    return jnp.pad(x, (0, _ROWS * _LANES - _N)).reshape(_ROWS, _LANES)


@jax.jit
def kernel(cls_logits, bbox_pred, anchors, image_h, image_w):
    planes = [
        _plane(cls_logits[0, :, 0]), _plane(cls_logits[0, :, 1]),
        _plane(bbox_pred[0, :, 0]), _plane(bbox_pred[0, :, 1]),
        _plane(bbox_pred[0, :, 2]), _plane(bbox_pred[0, :, 3]),
        _plane(anchors[:, 0]), _plane(anchors[:, 1]),
        _plane(anchors[:, 2]), _plane(anchors[:, 3]),
    ]
    dims = jnp.stack([image_w, image_h]).astype(jnp.float32).reshape(1, 2)

    out = pl.pallas_call(
        _nms_body,
        out_shape=jax.ShapeDtypeStruct((8, _LANES), jnp.float32),
        in_specs=[pl.BlockSpec(memory_space=pltpu.VMEM)
                  for _ in range(10)] +
                 [pl.BlockSpec(memory_space=pltpu.SMEM)],
        out_specs=pl.BlockSpec(memory_space=pltpu.VMEM),
        scratch_shapes=[pltpu.VMEM((_ROWS, _LANES), jnp.float32)
                        for _ in range(5)],
    )(*planes, dims)

    return out[:5, :_MAXOUT].T


# speedup vs baseline: 14.5948x; 1.4401x over previous
import functools

import jax
import jax.numpy as jnp
from jax import lax
from jax.experimental import pallas as pl
from jax.experimental.pallas import tpu as pltpu
from jax.experimental.pallas import tpu_sc as plsc

_CV = 0.1
_SV = 0.2
_CONF = 0.01
_NMS_T = 0.5
_MAXOUT = 100
_N = 20000
_NP = 20480
_NW = 16
_CHUNK = _NP // _NW
_VPC = _CHUNK // 16
_NBLK = _NP // 16
_NSUP = _NBLK // 16
_SELP = 112
_NEG = -1e9
_BIG = 1 << 30


def _iota16():
    return lax.broadcasted_iota(jnp.int32, (16,), 0)


def _lane(v, lane_idx, fill):
    return jnp.max(jnp.where(_iota16() == lane_idx, v, fill))


def _sc_body(l0h, l1h, tch, tcyh, twh, thh, acxh, acyh, awh, ahh, dimh,
             out_hbm,
             b0, b1, b2, b3, b4, b5, b6, b7, b8, b9, dimv, fillb,
             fsc, fx0, fy0, fx1, fy1, bmax, smax,
             sx0, sy0, sx1, sy1, ssc,
             sh0, sh1, sh2, sh3, sh4):
    w = lax.axis_index("s")
    base = w * _CHUNK
    it = _iota16()

    for ref, hbm in ((b0, l0h), (b1, l1h), (b2, tch), (b3, tcyh), (b4, twh),
                     (b5, thh), (b6, acxh), (b7, acyh), (b8, awh), (b9, ahh)):
        pltpu.sync_copy(hbm.at[pl.ds(base, _CHUNK)], ref)
    pltpu.sync_copy(dimh, dimv)
    dv = dimv[...]
    sw = jnp.max(jnp.where(it == 0, dv, _NEG))
    sh = jnp.max(jnp.where(it == 1, dv, _NEG))

    a0 = b0[pl.ds(0, 16)]
    a1 = b1[pl.ds(0, 16)]
    mx0 = jnp.maximum(a0, a1)
    fillb[...] = jnp.exp(a1 - mx0) / (jnp.exp(a0 - mx0) + jnp.exp(a1 - mx0))

    def decode(i, carry):
        off = pl.multiple_of(i * 16, 16)
        sl = pl.ds(off, 16)
        a = b0[sl]
        b = b1[sl]
        tcx = b2[sl]
        tcy = b3[sl]
        tw = b4[sl]
        th = b5[sl]
        acx = b6[sl]
        acy = b7[sl]
        aw = b8[sl]
        ah = b9[sl]
        mx = jnp.maximum(a, b)
        e0 = jnp.exp(a - mx)
        e1 = jnp.exp(b - mx)
        fg = e1 / (e0 + e1)
        cx = tcx * _CV * aw + acx
        cy = tcy * _CV * ah + acy
        bw = jnp.exp(tw * _SV) * aw
        bh = jnp.exp(th * _SV) * ah
        gidx = base + off + it
        masked = jnp.where((fg > _CONF) & (gidx < _N), fg, _NEG)
        b0[sl] = masked
        b1[sl] = (cx - bw / 2.0) * sw
        b2[sl] = (cy - bh / 2.0) * sh
        b3[sl] = (cx + bw / 2.0) * sw
        b4[sl] = (cy + bh / 2.0) * sh
        return carry

    lax.fori_loop(0, _VPC, decode, 0)

    for ref, shr in ((b0, sh0), (b1, sh1), (b2, sh2), (b3, sh3), (b4, sh4)):
        pltpu.sync_copy(ref, shr.at[pl.ds(base, _CHUNK)])
    plsc.subcore_barrier()

    @pl.when(w == 0)
    def _phase_b():
        for shr, ref in ((sh0, fsc), (sh1, fx0), (sh2, fy0), (sh3, fx1),
                         (sh4, fy1)):
            pltpu.sync_copy(shr, ref)

        def mk_bmax(bi, carry):
            gbase = bi * 256
            acc = plsc.load_gather(fsc, [gbase + it * 16])
            for j in range(1, 16):
                acc = jnp.maximum(acc,
                                  plsc.load_gather(fsc, [gbase + it * 16 + j]))
            bmax[pl.ds(pl.multiple_of(bi * 16, 16), 16)] = acc
            return carry

        lax.fori_loop(0, _NSUP, mk_bmax, 0)

        for si in range(_NSUP // 16):
            gbase = si * 256
            acc = plsc.load_gather(bmax, [gbase + it * 16])
            for j in range(1, 16):
                acc = jnp.maximum(acc,
                                  plsc.load_gather(bmax, [gbase + it * 16 + j]))
            smax[pl.ds(si * 16, 16)] = acc

        c0 = _lane(fx0[pl.ds(0, 16)], 0, _NEG)
        c1 = _lane(fy0[pl.ds(0, 16)], 0, _NEG)
        c2 = _lane(fx1[pl.ds(0, 16)], 0, _NEG)
        c3 = _lane(fy1[pl.ds(0, 16)], 0, _NEG)
        c4 = _lane(fillb[...], 0, _NEG)
        for j in range(_SELP // 16):
            sl = pl.ds(j * 16, 16)
            sx0[sl] = jnp.broadcast_to(c0, (16,))
            sy0[sl] = jnp.broadcast_to(c1, (16,))
            sx1[sl] = jnp.broadcast_to(c2, (16,))
            sy1[sl] = jnp.broadcast_to(c3, (16,))
            ssc[sl] = jnp.broadcast_to(c4, (16,))

        def global_max():
            acc = smax[pl.ds(0, 16)]
            for si in range(1, _NSUP // 16):
                acc = jnp.maximum(acc, smax[pl.ds(si * 16, 16)])
            return jnp.max(acc)

        def cond(state):
            k, m = state
            return jnp.logical_and(k < _MAXOUT, m > _NEG)

        def body(state):
            k, m = state
            sacc = jnp.full((16,), _BIG, jnp.int32)
            for si in range(_NSUP // 16):
                v = smax[pl.ds(si * 16, 16)]
                sacc = jnp.minimum(sacc,
                                   jnp.where(v == m, si * 16 + it, _BIG))
            s = jnp.min(sacc)
            bv = bmax[pl.ds(pl.multiple_of(s * 16, 16), 16)]
            bnum = jnp.min(jnp.where(bv == m, s * 16 + it, _BIG))
            eoff = pl.multiple_of(bnum * 16, 16)
            ev = fsc[pl.ds(eoff, 16)]
            lane_g = jnp.min(jnp.where(ev == m, it, _BIG))

            hit = it == lane_g
            cx0 = jnp.max(jnp.where(hit, fx0[pl.ds(eoff, 16)], _NEG))
            cy0 = jnp.max(jnp.where(hit, fy0[pl.ds(eoff, 16)], _NEG))
            cx1 = jnp.max(jnp.where(hit, fx1[pl.ds(eoff, 16)], _NEG))
            cy1 = jnp.max(jnp.where(hit, fy1[pl.ds(eoff, 16)], _NEG))
            a1 = (jnp.maximum(cx1 - cx0, 0.0) *
                  jnp.maximum(cy1 - cy0, 0.0))

            def iou_step(j, acc):
                off = pl.multiple_of(j * 16, 16)
                tx0 = sx0[pl.ds(off, 16)]
                ty0 = sy0[pl.ds(off, 16)]
                tx1 = sx1[pl.ds(off, 16)]
                ty1 = sy1[pl.ds(off, 16)]
                iw = jnp.maximum(
                    jnp.minimum(cx1, tx1) - jnp.maximum(cx0, tx0), 0.0)
                ih = jnp.maximum(
                    jnp.minimum(cy1, ty1) - jnp.maximum(cy0, ty0), 0.0)
                inter = iw * ih
                a2 = (jnp.maximum(tx1 - tx0, 0.0) *
                      jnp.maximum(ty1 - ty0, 0.0))
                iou = inter / (a1 + a2 - inter + 1e-9)
                valid = (j * 16 + it) < k
                return jnp.maximum(acc, jnp.where(valid, iou, 0.0))

            nv = (k + 15) // 16
            iou_max = lax.fori_loop(0, nv, iou_step,
                                    jnp.zeros((16,), jnp.float32))
            supp = jnp.max(iou_max) > _NMS_T

            @pl.when(jnp.logical_not(supp))
            def _insert():
                koff = pl.multiple_of((k // 16) * 16, 16)
                klane = k & 15
                put = it == klane
                ksl = pl.ds(koff, 16)
                sx0[ksl] = jnp.where(put, cx0, sx0[ksl])
                sy0[ksl] = jnp.where(put, cy0, sy0[ksl])
                sx1[ksl] = jnp.where(put, cx1, sx1[ksl])
                sy1[ksl] = jnp.where(put, cy1, sy1[ksl])
                ssc[ksl] = jnp.where(put, m, ssc[ksl])

            k = k + jnp.where(supp, 0, 1).astype(jnp.int32)

            nev = jnp.where(hit, _NEG, ev)
            fsc[pl.ds(eoff, 16)] = nev
            nbv = jnp.where(it == (bnum & 15), jnp.max(nev), bv)
            bmax[pl.ds(pl.multiple_of(s * 16, 16), 16)] = nbv
            soff = pl.multiple_of((s // 16) * 16, 16)
            ssl = pl.ds(soff, 16)
            nsv = jnp.where(it == (s & 15), jnp.max(nbv), smax[ssl])
            smax[ssl] = nsv
            return k, global_max()

        lax.while_loop(cond, body, (jnp.int32(0), global_max()))

        pltpu.sync_copy(sx0, out_hbm.at[pl.ds(0 * _SELP, _SELP)])
        pltpu.sync_copy(sy0, out_hbm.at[pl.ds(1 * _SELP, _SELP)])
        pltpu.sync_copy(sx1, out_hbm.at[pl.ds(2 * _SELP, _SELP)])
        pltpu.sync_copy(sy1, out_hbm.at[pl.ds(3 * _SELP, _SELP)])
        pltpu.sync_copy(ssc, out_hbm.at[pl.ds(4 * _SELP, _SELP)])


def _plane(x):
    return jnp.pad(x, (0, _NP - _N))


@jax.jit
def kernel(cls_logits, bbox_pred, anchors, image_h, image_w):
    planes = [
        _plane(cls_logits[0, :, 0]), _plane(cls_logits[0, :, 1]),
        _plane(bbox_pred[0, :, 0]), _plane(bbox_pred[0, :, 1]),
        _plane(bbox_pred[0, :, 2]), _plane(bbox_pred[0, :, 3]),
        _plane(anchors[:, 0]), _plane(anchors[:, 1]),
        _plane(anchors[:, 2]), _plane(anchors[:, 3]),
    ]
    dims = jnp.zeros((16,), jnp.float32)
    dims = dims.at[0].set(jnp.float32(image_w)).at[1].set(jnp.float32(image_h))

    mesh = plsc.VectorSubcoreMesh(core_axis_name="c", subcore_axis_name="s",
                                  num_cores=1)
    run = pl.kernel(
        _sc_body,
        out_type=jax.ShapeDtypeStruct((5 * _SELP,), jnp.float32),
        mesh=mesh,
        compiler_params=pltpu.CompilerParams(needs_layout_passes=False),
        scratch_types=(
            [pltpu.VMEM((_CHUNK,), jnp.float32) for _ in range(10)] +
            [pltpu.VMEM((16,), jnp.float32), pltpu.VMEM((16,), jnp.float32)] +
            [pltpu.VMEM((_NP,), jnp.float32) for _ in range(5)] +
            [pltpu.VMEM((_NBLK,), jnp.float32),
             pltpu.VMEM((_NSUP,), jnp.float32)] +
            [pltpu.VMEM((_SELP,), jnp.float32) for _ in range(5)] +
            [pltpu.VMEM_SHARED((_NP,), jnp.float32) for _ in range(5)]
        ),
    )
    out = run(*planes, dims)
    return out.reshape(5, _SELP)[:, :_MAXOUT].T
